# Initial kernel scaffold; baseline (speedup 1.0000x reference)
import jax
import jax.numpy as jnp
from jax.experimental import pallas as pl


def kernel(object_features, language_embedding, centers, sizes, W1, b1, W2, b2):
    raise NotImplementedError("write your pallas kernel here")


# speedup vs baseline: 8.6946x; 8.6946x over previous
import functools

import jax
import jax.numpy as jnp
from jax import lax
from jax.experimental import pallas as pl
from jax.experimental.pallas import tpu as pltpu
from jax.experimental.pallas import tpu_sc as plsc

_B, _N, _D = 4, 1024, 320
_DL, _H, _K = 256, 256, 8
_R1 = 256
_R3 = 256
_NC, _NS = 2, 16
_NW = _NC * _NS
_GCHUNK = 128
_CW = 640


def _stage1_body(cen_ref, cenT_ref, of_ref, siz_ref, lang_ref,
                 w1at_ref, w1bt_ref, wgct_ref, wgst_ref, w1lt_ref, b1_ref,
                 idx_ref, gidx_ref, a_ref, cof_ref):
    b = pl.program_id(0)
    i = pl.program_id(1)
    cen = cen_ref[0]
    cenT = cenT_ref[0]
    d = jnp.zeros((_R1, _N), jnp.float32)
    for c in range(3):
        diff = cen[:, c:c + 1] - cenT[c:c + 1, :]
        d = d + diff * diff
    rowg = i * _R1 + lax.broadcasted_iota(jnp.int32, (_R1, _N), 0)
    colid = lax.broadcasted_iota(jnp.int32, (_R1, _N), 1)
    d = jnp.where(rowg == colid, jnp.inf, d)
    sels = []
    for _ in range(_K):
        m = jnp.min(d, axis=1, keepdims=True)
        sel = jnp.min(jnp.where(d == m, colid, _N), axis=1, keepdims=True)
        sels.append(sel)
        d = jnp.where(colid == sel, jnp.inf, d)
    idx = jnp.concatenate(sels, axis=1)
    idx_ref[0] = idx
    gidx_ref[0] = idx + b * _N

    of = of_ref[0]
    g = (jnp.dot(cen, wgct_ref[...], preferred_element_type=jnp.float32)
         * (1.0 / (5.0 + 1e-06))
         + jnp.dot(siz_ref[0], wgst_ref[...], preferred_element_type=jnp.float32)
         * 0.5)
    langp = jnp.dot(lang_ref[0], w1lt_ref[...],
                    preferred_element_type=jnp.float32)
    a_ref[0] = (jnp.dot(of, w1at_ref[...], preferred_element_type=jnp.float32)
                + langp + b1_ref[...] + g)
    cvals = (jnp.dot(of, w1bt_ref[...], preferred_element_type=jnp.float32)
             - g)
    cof_ref[0] = jnp.concatenate(
        [cvals, of, jnp.zeros((_R1, _CW - _H - _D), jnp.float32)], axis=1)


def _stage1(cen, cenT, of, siz, lang, w1at, w1bt, wgct, wgst, w1lt, b1):
    grid = (_B, _N // _R1)
    return pl.pallas_call(
        _stage1_body,
        grid=grid,
        in_specs=[
            pl.BlockSpec((1, _R1, 3), lambda b, i: (b, i, 0)),
            pl.BlockSpec((1, 3, _N), lambda b, i: (b, 0, 0)),
            pl.BlockSpec((1, _R1, _D), lambda b, i: (b, i, 0)),
            pl.BlockSpec((1, _R1, 3), lambda b, i: (b, i, 0)),
            pl.BlockSpec((1, 1, _DL), lambda b, i: (b, 0, 0)),
            pl.BlockSpec((_D, _H), lambda b, i: (0, 0)),
            pl.BlockSpec((_D, _H), lambda b, i: (0, 0)),
            pl.BlockSpec((3, _H), lambda b, i: (0, 0)),
            pl.BlockSpec((3, _H), lambda b, i: (0, 0)),
            pl.BlockSpec((_DL, _H), lambda b, i: (0, 0)),
            pl.BlockSpec((1, _H), lambda b, i: (0, 0)),
        ],
        out_specs=[
            pl.BlockSpec((1, _R1, _K), lambda b, i: (b, i, 0)),
            pl.BlockSpec((1, _R1, _K), lambda b, i: (b, i, 0)),
            pl.BlockSpec((1, _R1, _H), lambda b, i: (b, i, 0)),
            pl.BlockSpec((1, _R1, _CW), lambda b, i: (b, i, 0)),
        ],
        out_shape=[
            jax.ShapeDtypeStruct((_B, _N, _K), jnp.int32),
            jax.ShapeDtypeStruct((_B, _N, _K), jnp.int32),
            jax.ShapeDtypeStruct((_B, _N, _H), jnp.float32),
            jax.ShapeDtypeStruct((_B, _N, _CW), jnp.float32),
        ],
    )(cen, cenT, of, siz, lang, w1at, w1bt, wgct, wgst, w1lt, b1)


def _sc_gather(gidx_flat, cof_all):
    n_idx = _B * _N * _K
    per_w = n_idx // _NW
    n_chunks = per_w // _GCHUNK

    mesh = plsc.VectorSubcoreMesh(core_axis_name="c", subcore_axis_name="s")

    @functools.partial(
        pl.kernel,
        mesh=mesh,
        out_type=jax.ShapeDtypeStruct((n_idx, _CW), jnp.float32),
        scratch_types=[
            pltpu.VMEM((_GCHUNK,), jnp.int32),
            pltpu.VMEM((_GCHUNK, _CW), jnp.float32),
            pltpu.SemaphoreType.DMA,
        ],
    )
    def k(gidx_hbm, cof_hbm, out_hbm, idx_v, rows_v, sem):
        wid = lax.axis_index("s") * _NC + lax.axis_index("c")
        for t in range(n_chunks):
            base = wid * per_w + t * _GCHUNK
            pltpu.sync_copy(gidx_hbm.at[pl.ds(base, _GCHUNK)], idx_v)
            pltpu.async_copy(cof_hbm.at[idx_v], rows_v, sem).wait()
            pltpu.sync_copy(rows_v, out_hbm.at[pl.ds(base, _GCHUNK)])

    return k(gidx_flat, cof_all)


def _stage3_body(a_ref, cofg_ref, of_ref, w2t_ref, b2_ref, out_ref, w_ref):
    a = a_ref[...]
    arep = jnp.concatenate([a] * _K, axis=1)
    cofg = cofg_ref[...]
    cg = jnp.concatenate(
        [cofg[:, k * _CW:k * _CW + _H] for k in range(_K)], axis=1)
    h = jnp.maximum(arep + cg, 0.0)
    scores = (jnp.dot(h, w2t_ref[...], preferred_element_type=jnp.float32)
              + b2_ref[...])
    m = jnp.max(scores, axis=1, keepdims=True)
    e = jnp.exp(scores - m)
    w = e / jnp.sum(e, axis=1, keepdims=True)
    acc = of_ref[...]
    for k in range(_K):
        acc = acc + w[:, k:k + 1] * cofg[:, k * _CW + _H:k * _CW + _H + _D]
    out_ref[...] = acc
    w_ref[...] = w


def _stage3(a_flat, cofg, of_flat, w2t, b2):
    M = _B * _N
    grid = (M // _R3,)
    return pl.pallas_call(
        _stage3_body,
        grid=grid,
        in_specs=[
            pl.BlockSpec((_R3, _H), lambda i: (i, 0)),
            pl.BlockSpec((_R3, _K * _CW), lambda i: (i, 0)),
            pl.BlockSpec((_R3, _D), lambda i: (i, 0)),
            pl.BlockSpec((_K * _H, _K), lambda i: (0, 0)),
            pl.BlockSpec((1, _K), lambda i: (0, 0)),
        ],
        out_specs=[
            pl.BlockSpec((_R3, _D), lambda i: (i, 0)),
            pl.BlockSpec((_R3, _K), lambda i: (i, 0)),
        ],
        out_shape=[
            jax.ShapeDtypeStruct((M, _D), jnp.float32),
            jax.ShapeDtypeStruct((M, _K), jnp.float32),
        ],
    )(a_flat, cofg, of_flat, w2t, b2)


def kernel(object_features, language_embedding, centers, sizes, W1, b1, W2, b2):
    w1at = W1[:, :_D].T
    w1bt = W1[:, _D:2 * _D].T
    wgct = W1[:, 2 * _D:2 * _D + 3].T
    wgst = W1[:, 2 * _D + 3:2 * _D + 6].T
    w1lt = W1[:, 2 * _D + 6:].T
    b1r = b1.reshape(1, _H)
    cenT = jnp.transpose(centers, (0, 2, 1))
    w2t = (jnp.eye(_K, dtype=jnp.float32)[:, None, :]
           * W2[0][None, :, None]).reshape(_K * _H, _K)
    b2r = jnp.broadcast_to(b2.reshape(1, 1), (1, _K))

    idx, gidx, a_all, cof_all = _stage1(
        centers, cenT, object_features, sizes,
        language_embedding.reshape(_B, 1, _DL),
        w1at, w1bt, wgct, wgst, w1lt, b1r)

    of_flat = object_features.reshape(_B * _N, _D)
    cofg = _sc_gather(gidx.reshape(-1), cof_all.reshape(_B * _N, _CW))

    out, w = _stage3(
        a_all.reshape(_B * _N, _H),
        cofg.reshape(_B * _N, _K * _CW),
        of_flat, w2t, b2r)

    return (out.reshape(_B, _N, _D), w.reshape(_B, _N, _K), idx)


# speedup vs baseline: 12.0536x; 1.3863x over previous
import functools

import jax
import jax.numpy as jnp
from jax import lax
from jax.experimental import pallas as pl
from jax.experimental.pallas import tpu as pltpu
from jax.experimental.pallas import tpu_sc as plsc

_B, _N, _D = 4, 1024, 320
_DL, _H, _K = 256, 256, 8
_R1 = 256
_R3 = 256
_NC, _NS = 2, 16
_NW = _NC * _NS
_GCHUNK = 128
_CW = 640


def _stage1_body(cen_ref, cenT_ref, of_ref, siz_ref, lang_ref,
                 w1at_ref, w1bt_ref, wgct_ref, wgst_ref, w1lt_ref, b1_ref,
                 idx_ref, gidx_ref, a_ref, c_ref):
    b = pl.program_id(0)
    i = pl.program_id(1)
    cen = cen_ref[0]
    cenT = cenT_ref[0]
    d = jnp.zeros((_R1, _N), jnp.float32)
    for c in range(3):
        diff = cen[:, c:c + 1] - cenT[c:c + 1, :]
        d = d + diff * diff
    rowg = i * _R1 + lax.broadcasted_iota(jnp.int32, (_R1, _N), 0)
    colid = lax.broadcasted_iota(jnp.int32, (_R1, _N), 1)
    d = jnp.where(rowg == colid, jnp.inf, d)
    sels = []
    for _ in range(_K):
        m = jnp.min(d, axis=1, keepdims=True)
        sel = jnp.min(jnp.where(d == m, colid, _N), axis=1, keepdims=True)
        sels.append(sel)
        d = jnp.where(colid == sel, jnp.inf, d)
    idx = jnp.concatenate(sels, axis=1)
    idx_ref[0] = idx
    gidx_ref[0] = idx + b * _N

    of = of_ref[0]
    g = (jnp.dot(cen, wgct_ref[...], preferred_element_type=jnp.float32)
         * (1.0 / (5.0 + 1e-06))
         + jnp.dot(siz_ref[0], wgst_ref[...], preferred_element_type=jnp.float32)
         * 0.5)
    langp = jnp.dot(lang_ref[0], w1lt_ref[...],
                    preferred_element_type=jnp.float32)
    a_ref[0] = (jnp.dot(of, w1at_ref[...], preferred_element_type=jnp.float32)
                + langp + b1_ref[...] + g)
    c_ref[0] = (jnp.dot(of, w1bt_ref[...], preferred_element_type=jnp.float32)
                - g)


def _stage1(cen, cenT, of, siz, lang, w1at, w1bt, wgct, wgst, w1lt, b1):
    grid = (_B, _N // _R1)
    return pl.pallas_call(
        _stage1_body,
        grid=grid,
        in_specs=[
            pl.BlockSpec((1, _R1, 3), lambda b, i: (b, i, 0)),
            pl.BlockSpec((1, 3, _N), lambda b, i: (b, 0, 0)),
            pl.BlockSpec((1, _R1, _D), lambda b, i: (b, i, 0)),
            pl.BlockSpec((1, _R1, 3), lambda b, i: (b, i, 0)),
            pl.BlockSpec((1, 1, _DL), lambda b, i: (b, 0, 0)),
            pl.BlockSpec((_D, _H), lambda b, i: (0, 0)),
            pl.BlockSpec((_D, _H), lambda b, i: (0, 0)),
            pl.BlockSpec((3, _H), lambda b, i: (0, 0)),
            pl.BlockSpec((3, _H), lambda b, i: (0, 0)),
            pl.BlockSpec((_DL, _H), lambda b, i: (0, 0)),
            pl.BlockSpec((1, _H), lambda b, i: (0, 0)),
        ],
        out_specs=[
            pl.BlockSpec((1, _R1, _K), lambda b, i: (b, i, 0)),
            pl.BlockSpec((1, _R1, _K), lambda b, i: (b, i, 0)),
            pl.BlockSpec((1, _R1, _H), lambda b, i: (b, i, 0)),
            pl.BlockSpec((1, _R1, _H), lambda b, i: (b, i, 0)),
        ],
        out_shape=[
            jax.ShapeDtypeStruct((_B, _N, _K), jnp.int32),
            jax.ShapeDtypeStruct((_B, _N, _K), jnp.int32),
            jax.ShapeDtypeStruct((_B, _N, _H), jnp.float32),
            jax.ShapeDtypeStruct((_B, _N, _H), jnp.float32),
        ],
    )(cen, cenT, of, siz, lang, w1at, w1bt, wgct, wgst, w1lt, b1)


def _sc_gather(gidx_flat, c_all):
    n_idx = _B * _N * _K
    per_w = n_idx // _NW
    n_chunks = per_w // _GCHUNK

    mesh = plsc.VectorSubcoreMesh(core_axis_name="c", subcore_axis_name="s")

    @functools.partial(
        pl.kernel,
        mesh=mesh,
        out_type=jax.ShapeDtypeStruct((n_idx, _H), jnp.float32),
        scratch_types=[
            pltpu.VMEM((per_w,), jnp.int32),
            pltpu.VMEM((2, _GCHUNK, _H), jnp.float32),
            pltpu.SemaphoreType.DMA,
        ],
    )
    def k(gidx_hbm, c_hbm, out_hbm, idx_v, rows_v, gsem):
        wid = lax.axis_index("s") * _NC + lax.axis_index("c")
        base0 = wid * per_w
        pltpu.sync_copy(gidx_hbm.at[pl.ds(base0, per_w)], idx_v)
        cps = [None, None]
        cps[0] = pltpu.async_copy(
            c_hbm.at[idx_v.at[pl.ds(0, _GCHUNK)]], rows_v.at[0], gsem)
        for t in range(n_chunks):
            s = t % 2
            if t + 1 < n_chunks:
                cps[1 - s] = pltpu.async_copy(
                    c_hbm.at[idx_v.at[pl.ds((t + 1) * _GCHUNK, _GCHUNK)]],
                    rows_v.at[1 - s], gsem)
            cps[s].wait()
            pltpu.sync_copy(
                rows_v.at[s], out_hbm.at[pl.ds(base0 + t * _GCHUNK, _GCHUNK)])

    return k(gidx_flat, c_all)


def _stage3_body(a_ref, cg_ref, idx_ref, offull_ref, of_ref, w2t_ref, b2_ref,
                 out_ref, w_ref):
    a = a_ref[0]
    arep = jnp.concatenate([a] * _K, axis=1)
    h = jnp.maximum(arep + cg_ref[0], 0.0)
    scores = (jnp.dot(h, w2t_ref[...], preferred_element_type=jnp.float32)
              + b2_ref[...])
    m = jnp.max(scores, axis=1, keepdims=True)
    e = jnp.exp(scores - m)
    w = e / jnp.sum(e, axis=1, keepdims=True)
    idx = idx_ref[0]
    colid = lax.broadcasted_iota(jnp.int32, (_R3, _N), 1)
    wmat = jnp.zeros((_R3, _N), jnp.float32)
    for k in range(_K):
        wmat = wmat + jnp.where(colid == idx[:, k:k + 1], w[:, k:k + 1], 0.0)
    ctx = jnp.dot(wmat, offull_ref[0], preferred_element_type=jnp.float32)
    out_ref[0] = of_ref[0] + ctx
    w_ref[0] = w


def _stage3(a_all, cg, idx, of, w2t, b2):
    grid = (_B, _N // _R3)
    return pl.pallas_call(
        _stage3_body,
        grid=grid,
        in_specs=[
            pl.BlockSpec((1, _R3, _H), lambda b, i: (b, i, 0)),
            pl.BlockSpec((1, _R3, _K * _H), lambda b, i: (b, i, 0)),
            pl.BlockSpec((1, _R3, _K), lambda b, i: (b, i, 0)),
            pl.BlockSpec((1, _N, _D), lambda b, i: (b, 0, 0)),
            pl.BlockSpec((1, _R3, _D), lambda b, i: (b, i, 0)),
            pl.BlockSpec((_K * _H, _K), lambda b, i: (0, 0)),
            pl.BlockSpec((1, _K), lambda b, i: (0, 0)),
        ],
        out_specs=[
            pl.BlockSpec((1, _R3, _D), lambda b, i: (b, i, 0)),
            pl.BlockSpec((1, _R3, _K), lambda b, i: (b, i, 0)),
        ],
        out_shape=[
            jax.ShapeDtypeStruct((_B, _N, _D), jnp.float32),
            jax.ShapeDtypeStruct((_B, _N, _K), jnp.float32),
        ],
    )(a_all, cg, idx, of, of, w2t, b2)


def kernel(object_features, language_embedding, centers, sizes, W1, b1, W2, b2):
    w1at = W1[:, :_D].T
    w1bt = W1[:, _D:2 * _D].T
    wgct = W1[:, 2 * _D:2 * _D + 3].T
    wgst = W1[:, 2 * _D + 3:2 * _D + 6].T
    w1lt = W1[:, 2 * _D + 6:].T
    b1r = b1.reshape(1, _H)
    cenT = jnp.transpose(centers, (0, 2, 1))
    w2t = (jnp.eye(_K, dtype=jnp.float32)[:, None, :]
           * W2[0][None, :, None]).reshape(_K * _H, _K)
    b2r = jnp.broadcast_to(b2.reshape(1, 1), (1, _K))

    idx, gidx, a_all, c_all = _stage1(
        centers, cenT, object_features, sizes,
        language_embedding.reshape(_B, 1, _DL),
        w1at, w1bt, wgct, wgst, w1lt, b1r)

    cg = _sc_gather(gidx.reshape(-1), c_all.reshape(_B * _N, _H))

    out, w = _stage3(
        a_all, cg.reshape(_B, _N, _K * _H), idx, object_features, w2t, b2r)

    return (out, w, idx)


# speedup vs baseline: 13.9158x; 1.1545x over previous
import functools

import jax
import jax.numpy as jnp
from jax import lax
from jax.experimental import pallas as pl
from jax.experimental.pallas import tpu as pltpu
from jax.experimental.pallas import tpu_sc as plsc

_B, _N, _D = 4, 1024, 320
_DL, _H, _K = 256, 256, 8
_R1 = 256
_R3 = 256
_NC, _NS = 2, 16
_NW = _NC * _NS
_GCHUNK = 128
_CW = 640


def _dot_t(x, w):
    return lax.dot_general(x, w, (((1,), (1,)), ((), ())),
                           preferred_element_type=jnp.float32)


def _stage1_body(cen_ref, cenall_ref, of_ref, siz_ref, lang_ref,
                 w1a_ref, w1b_ref, wgc_ref, wgs_ref, w1l_ref, b1_ref,
                 eye3_ref, idx_ref, gidxt_ref, a_ref, c_ref):
    b = pl.program_id(0)
    i = pl.program_id(1)
    cen = cen_ref[0]
    cenT = jnp.transpose(cenall_ref[0], (1, 0))
    d = jnp.zeros((_R1, _N), jnp.float32)
    for c in range(3):
        diff = cen[:, c:c + 1] - cenT[c:c + 1, :]
        d = d + diff * diff
    d = d / 25.0
    rowg = i * _R1 + lax.broadcasted_iota(jnp.int32, (_R1, _N), 0)
    colid = lax.broadcasted_iota(jnp.int32, (_R1, _N), 1)
    d = jnp.where(rowg == colid, jnp.inf, d)
    sels = []
    for k in range(_K):
        m = jnp.min(d, axis=1, keepdims=True)
        sel = jnp.min(jnp.where(d == m, colid, _N), axis=1, keepdims=True)
        sels.append(sel)
        d = jnp.where(colid == sel, jnp.inf, d)
        gidxt_ref[k, :] = sel[:, 0] + b * _N
    idx_ref[0] = jnp.concatenate(sels, axis=1)

    of = of_ref[0]
    g = (_dot_t(cen, wgc_ref[...]) * (1.0 / (5.0 + 1e-06))
         + _dot_t(siz_ref[0], wgs_ref[...]) * 0.5)
    langp = _dot_t(lang_ref[0], w1l_ref[...])
    a_ref[0] = _dot_t(of, w1a_ref[...]) + langp + b1_ref[...] + g
    c_ref[0] = _dot_t(of, w1b_ref[...]) - g


def _stage1(cen, of, siz, lang, w1a, w1b, wgc, wgs, w1l, b1, eye3):
    grid = (_B, _N // _R1)
    return pl.pallas_call(
        _stage1_body,
        grid=grid,
        in_specs=[
            pl.BlockSpec((1, _R1, 3), lambda b, i: (b, i, 0)),
            pl.BlockSpec((1, _N, 3), lambda b, i: (b, 0, 0)),
            pl.BlockSpec((1, _R1, _D), lambda b, i: (b, i, 0)),
            pl.BlockSpec((1, _R1, 3), lambda b, i: (b, i, 0)),
            pl.BlockSpec((1, 1, _DL), lambda b, i: (b, 0, 0)),
            pl.BlockSpec((_H, _D), lambda b, i: (0, 0)),
            pl.BlockSpec((_H, _D), lambda b, i: (0, 0)),
            pl.BlockSpec((_H, 3), lambda b, i: (0, 0)),
            pl.BlockSpec((_H, 3), lambda b, i: (0, 0)),
            pl.BlockSpec((_H, _DL), lambda b, i: (0, 0)),
            pl.BlockSpec((1, _H), lambda b, i: (0, 0)),
            pl.BlockSpec((3, 3), lambda b, i: (0, 0)),
        ],
        out_specs=[
            pl.BlockSpec((1, _R1, _K), lambda b, i: (b, i, 0)),
            pl.BlockSpec((_K, _R1), lambda b, i: (0, b * (_N // _R1) + i)),
            pl.BlockSpec((1, _R1, _H), lambda b, i: (b, i, 0)),
            pl.BlockSpec((1, _R1, _H), lambda b, i: (b, i, 0)),
        ],
        out_shape=[
            jax.ShapeDtypeStruct((_B, _N, _K), jnp.int32),
            jax.ShapeDtypeStruct((_K, _B * _N), jnp.int32),
            jax.ShapeDtypeStruct((_B, _N, _H), jnp.float32),
            jax.ShapeDtypeStruct((_B, _N, _H), jnp.float32),
        ],
    )(cen, cen, of, siz, lang, w1a, w1b, wgc, wgs, w1l, b1, eye3)


def _sc_gather(gidxt, c_all):
    n_idx = _B * _N * _K
    per_w = n_idx // _NW
    n_chunks = per_w // _GCHUNK

    mesh = plsc.VectorSubcoreMesh(core_axis_name="c", subcore_axis_name="s")

    @functools.partial(
        pl.kernel,
        mesh=mesh,
        out_type=jax.ShapeDtypeStruct((n_idx, _H), jnp.float32),
        scratch_types=[
            pltpu.VMEM((per_w,), jnp.int32),
            pltpu.VMEM((2, _GCHUNK, _H), jnp.float32),
            pltpu.SemaphoreType.DMA,
        ],
    )
    def k(gidx_hbm, c_hbm, out_hbm, idx_v, rows_v, gsem):
        wid = lax.axis_index("s") * _NC + lax.axis_index("c")
        base0 = wid * per_w
        pltpu.sync_copy(
            gidx_hbm.at[wid // _B, pl.ds((wid % _B) * per_w, per_w)], idx_v)
        cps = [None, None]
        cps[0] = pltpu.async_copy(
            c_hbm.at[idx_v.at[pl.ds(0, _GCHUNK)]], rows_v.at[0], gsem)
        for t in range(n_chunks):
            s = t % 2
            if t + 1 < n_chunks:
                cps[1 - s] = pltpu.async_copy(
                    c_hbm.at[idx_v.at[pl.ds((t + 1) * _GCHUNK, _GCHUNK)]],
                    rows_v.at[1 - s], gsem)
            cps[s].wait()
            pltpu.sync_copy(
                rows_v.at[s], out_hbm.at[pl.ds(base0 + t * _GCHUNK, _GCHUNK)])

    return k(gidxt, c_all)


def _stage3_body(a_ref, cg_ref, idx_ref, offull_ref, of_ref, w2_ref, b2_ref,
                 out_ref, w_ref):
    a = a_ref[0]
    cols = []
    for k in range(_K):
        h = jnp.maximum(a + cg_ref[k], 0.0)
        cols.append(_dot_t(h, w2_ref[...]))
    scores = jnp.concatenate(cols, axis=1) + b2_ref[...]
    m = jnp.max(scores, axis=1, keepdims=True)
    e = jnp.exp(scores - m)
    w = e / jnp.sum(e, axis=1, keepdims=True)
    idx = idx_ref[0]
    colid = lax.broadcasted_iota(jnp.int32, (_R3, _N), 1)
    wmat = jnp.zeros((_R3, _N), jnp.float32)
    for k in range(_K):
        wmat = wmat + jnp.where(colid == idx[:, k:k + 1], w[:, k:k + 1], 0.0)
    ctx = jnp.dot(wmat, offull_ref[0], preferred_element_type=jnp.float32)
    out_ref[0] = of_ref[0] + ctx
    w_ref[0] = w


def _stage3(a_all, cg3, idx, of, w2, b2):
    grid = (_B, _N // _R3)
    nb = _B * _N // _R3
    return pl.pallas_call(
        _stage3_body,
        grid=grid,
        in_specs=[
            pl.BlockSpec((1, _R3, _H), lambda b, i: (b, i, 0)),
            pl.BlockSpec((_K, _R3, _H),
                         lambda b, i: (0, b * (_N // _R3) + i, 0)),
            pl.BlockSpec((1, _R3, _K), lambda b, i: (b, i, 0)),
            pl.BlockSpec((1, _N, _D), lambda b, i: (b, 0, 0)),
            pl.BlockSpec((1, _R3, _D), lambda b, i: (b, i, 0)),
            pl.BlockSpec((1, _H), lambda b, i: (0, 0)),
            pl.BlockSpec((1, _K), lambda b, i: (0, 0)),
        ],
        out_specs=[
            pl.BlockSpec((1, _R3, _D), lambda b, i: (b, i, 0)),
            pl.BlockSpec((1, _R3, _K), lambda b, i: (b, i, 0)),
        ],
        out_shape=[
            jax.ShapeDtypeStruct((_B, _N, _D), jnp.float32),
            jax.ShapeDtypeStruct((_B, _N, _K), jnp.float32),
        ],
    )(a_all, cg3, idx, of, of, w2, b2)


def kernel(object_features, language_embedding, centers, sizes, W1, b1, W2, b2):
    w1a = W1[:, :_D]
    w1b = W1[:, _D:2 * _D]
    wgc = W1[:, 2 * _D:2 * _D + 3]
    wgs = W1[:, 2 * _D + 3:2 * _D + 6]
    w1l = W1[:, 2 * _D + 6:]
    b1r = b1.reshape(1, _H)
    b2r = jnp.broadcast_to(b2.reshape(1, 1), (1, _K))
    eye3 = jnp.eye(3, dtype=jnp.float32)

    idx, gidxt, a_all, c_all = _stage1(
        centers, object_features, sizes,
        language_embedding.reshape(_B, 1, _DL),
        w1a, w1b, wgc, wgs, w1l, b1r, eye3)

    cg = _sc_gather(gidxt, c_all.reshape(_B * _N, _H))

    out, w = _stage3(
        a_all, cg.reshape(_K, _B * _N, _H), idx, object_features, W2, b2r)

    return (out, w, idx)


# speedup vs baseline: 16.7993x; 1.2072x over previous
import functools

import jax
import jax.numpy as jnp
from jax import lax
from jax.experimental import pallas as pl
from jax.experimental.pallas import tpu as pltpu
from jax.experimental.pallas import tpu_sc as plsc

_B, _N, _D = 4, 1024, 320
_DL, _H, _K = 256, 256, 8
_R1 = 256
_R3 = 256
_NC, _NS = 2, 16
_NW = _NC * _NS
_GCHUNK = 128
_CW = 640


def _dot_t(x, w):
    return lax.dot_general(x, w, (((1,), (1,)), ((), ())),
                           preferred_element_type=jnp.float32)


def _stage1_body(cenall_ref, of_ref, siz_ref, lang_ref,
                 w1a_ref, w1b_ref, wgc_ref, wgs_ref, w1l_ref, b1_ref,
                 idx_ref, gidxt_ref, a_ref, c_ref):
    b = pl.program_id(0)
    i = pl.program_id(1)
    cen = cenall_ref[0, pl.ds(i * _R1, _R1), :]
    cenT = jnp.transpose(cenall_ref[0], (1, 0))
    d = jnp.zeros((_R1, _N), jnp.float32)
    for c in range(3):
        diff = cen[:, c:c + 1] - cenT[c:c + 1, :]
        d = d + diff * diff
    d = d / 25.0
    rowg = i * _R1 + lax.broadcasted_iota(jnp.int32, (_R1, _N), 0)
    colid_i = lax.broadcasted_iota(jnp.int32, (_R1, _N), 1)
    d = jnp.where(rowg == colid_i, jnp.inf, d)
    colid = colid_i.astype(jnp.float32)
    sels = []
    for k in range(_K):
        m = jnp.min(d, axis=1, keepdims=True)
        sel = jnp.min(jnp.where(d == m, colid, jnp.float32(_N)),
                      axis=1, keepdims=True)
        sels.append(sel)
        d = jnp.where(colid == sel, jnp.inf, d)
    idxf = jnp.concatenate(sels, axis=1)
    idx_ref[0] = idxf.astype(jnp.int32)
    gidxt_ref[...] = (jnp.transpose(idxf, (1, 0))
                      + jnp.float32(b * _N)).astype(jnp.int32)

    of = of_ref[0]
    g = (_dot_t(cen, wgc_ref[...]) * (1.0 / (5.0 + 1e-06))
         + _dot_t(siz_ref[0], wgs_ref[...]) * 0.5)
    langp = _dot_t(lang_ref[0], w1l_ref[...])
    a_ref[0] = _dot_t(of, w1a_ref[...]) + langp + b1_ref[...] + g
    c_ref[0] = _dot_t(of, w1b_ref[...]) - g


def _stage1(cen, of, siz, lang, w1a, w1b, wgc, wgs, w1l, b1):
    grid = (_B, _N // _R1)
    return pl.pallas_call(
        _stage1_body,
        grid=grid,
        in_specs=[
            pl.BlockSpec((1, _N, 3), lambda b, i: (b, 0, 0)),
            pl.BlockSpec((1, _R1, _D), lambda b, i: (b, i, 0)),
            pl.BlockSpec((1, _R1, 3), lambda b, i: (b, i, 0)),
            pl.BlockSpec((1, 1, _DL), lambda b, i: (b, 0, 0)),
            pl.BlockSpec((_H, _D), lambda b, i: (0, 0)),
            pl.BlockSpec((_H, _D), lambda b, i: (0, 0)),
            pl.BlockSpec((_H, 3), lambda b, i: (0, 0)),
            pl.BlockSpec((_H, 3), lambda b, i: (0, 0)),
            pl.BlockSpec((_H, _DL), lambda b, i: (0, 0)),
            pl.BlockSpec((1, _H), lambda b, i: (0, 0)),
        ],
        out_specs=[
            pl.BlockSpec((1, _R1, _K), lambda b, i: (b, i, 0)),
            pl.BlockSpec((_K, _R1), lambda b, i: (0, b * (_N // _R1) + i)),
            pl.BlockSpec((1, _R1, _H), lambda b, i: (b, i, 0)),
            pl.BlockSpec((1, _R1, _H), lambda b, i: (b, i, 0)),
        ],
        out_shape=[
            jax.ShapeDtypeStruct((_B, _N, _K), jnp.int32),
            jax.ShapeDtypeStruct((_K, _B * _N), jnp.int32),
            jax.ShapeDtypeStruct((_B, _N, _H), jnp.float32),
            jax.ShapeDtypeStruct((_B, _N, _H), jnp.float32),
        ],
    )(cen, of, siz, lang, w1a, w1b, wgc, wgs, w1l, b1)


def _sc_gather(gidxt, c_all):
    n_idx = _B * _N * _K
    per_w = n_idx // _NW
    n_chunks = per_w // _GCHUNK

    mesh = plsc.VectorSubcoreMesh(core_axis_name="c", subcore_axis_name="s")

    @functools.partial(
        pl.kernel,
        mesh=mesh,
        out_type=jax.ShapeDtypeStruct((n_idx, _H), jnp.float32),
        scratch_types=[
            pltpu.VMEM((per_w,), jnp.int32),
            pltpu.VMEM((2, _GCHUNK, _H), jnp.float32),
            pltpu.SemaphoreType.DMA,
        ],
    )
    def k(gidx_hbm, c_hbm, out_hbm, idx_v, rows_v, gsem):
        wid = lax.axis_index("s") * _NC + lax.axis_index("c")
        base0 = wid * per_w
        pltpu.sync_copy(
            gidx_hbm.at[wid // _B, pl.ds((wid % _B) * per_w, per_w)], idx_v)
        cps = [None, None]
        cps[0] = pltpu.async_copy(
            c_hbm.at[idx_v.at[pl.ds(0, _GCHUNK)]], rows_v.at[0], gsem)
        for t in range(n_chunks):
            s = t % 2
            if t + 1 < n_chunks:
                cps[1 - s] = pltpu.async_copy(
                    c_hbm.at[idx_v.at[pl.ds((t + 1) * _GCHUNK, _GCHUNK)]],
                    rows_v.at[1 - s], gsem)
            cps[s].wait()
            pltpu.sync_copy(
                rows_v.at[s], out_hbm.at[pl.ds(base0 + t * _GCHUNK, _GCHUNK)])

    return k(gidxt, c_all)


def _stage3_body(a_ref, cg_ref, idx_ref, offull_ref, w2_ref, b2_ref,
                 out_ref, w_ref):
    i = pl.program_id(1)
    a = a_ref[0]
    cols = []
    for k in range(_K):
        h = jnp.maximum(a + cg_ref[k], 0.0)
        cols.append(_dot_t(h, w2_ref[...]))
    scores = jnp.concatenate(cols, axis=1) + b2_ref[...]
    m = jnp.max(scores, axis=1, keepdims=True)
    e = jnp.exp(scores - m)
    w = e / jnp.sum(e, axis=1, keepdims=True)
    idx = idx_ref[0]
    colid = lax.broadcasted_iota(jnp.int32, (_R3, _N), 1)
    wmat = jnp.zeros((_R3, _N), jnp.float32)
    for k in range(_K):
        wmat = wmat + jnp.where(colid == idx[:, k:k + 1], w[:, k:k + 1], 0.0)
    ctx = jnp.dot(wmat, offull_ref[0], preferred_element_type=jnp.float32)
    out_ref[0] = offull_ref[0, pl.ds(i * _R3, _R3), :] + ctx
    w_ref[0] = w


def _stage3(a_all, cg3, idx, of, w2, b2):
    grid = (_B, _N // _R3)
    nb = _B * _N // _R3
    return pl.pallas_call(
        _stage3_body,
        grid=grid,
        in_specs=[
            pl.BlockSpec((1, _R3, _H), lambda b, i: (b, i, 0)),
            pl.BlockSpec((_K, _R3, _H),
                         lambda b, i: (0, b * (_N // _R3) + i, 0)),
            pl.BlockSpec((1, _R3, _K), lambda b, i: (b, i, 0)),
            pl.BlockSpec((1, _N, _D), lambda b, i: (b, 0, 0)),
            pl.BlockSpec((1, _H), lambda b, i: (0, 0)),
            pl.BlockSpec((1, _K), lambda b, i: (0, 0)),
        ],
        out_specs=[
            pl.BlockSpec((1, _R3, _D), lambda b, i: (b, i, 0)),
            pl.BlockSpec((1, _R3, _K), lambda b, i: (b, i, 0)),
        ],
        out_shape=[
            jax.ShapeDtypeStruct((_B, _N, _D), jnp.float32),
            jax.ShapeDtypeStruct((_B, _N, _K), jnp.float32),
        ],
    )(a_all, cg3, idx, of, w2, b2)


def kernel(object_features, language_embedding, centers, sizes, W1, b1, W2, b2):
    w1a = W1[:, :_D]
    w1b = W1[:, _D:2 * _D]
    wgc = W1[:, 2 * _D:2 * _D + 3]
    wgs = W1[:, 2 * _D + 3:2 * _D + 6]
    w1l = W1[:, 2 * _D + 6:]
    b1r = b1.reshape(1, _H)
    b2r = jnp.broadcast_to(b2.reshape(1, 1), (1, _K))

    idx, gidxt, a_all, c_all = _stage1(
        centers, object_features, sizes,
        language_embedding.reshape(_B, 1, _DL),
        w1a, w1b, wgc, wgs, w1l, b1r)

    cg = _sc_gather(gidxt, c_all.reshape(_B * _N, _H))

    out, w = _stage3(
        a_all, cg.reshape(_K, _B * _N, _H), idx, object_features, W2, b2r)

    return (out, w, idx)


# speedup vs baseline: 18.3286x; 1.0910x over previous
import functools

import jax
import jax.numpy as jnp
from jax import lax
from jax.experimental import pallas as pl
from jax.experimental.pallas import tpu as pltpu
from jax.experimental.pallas import tpu_sc as plsc

_B, _N, _D = 4, 1024, 320
_DL, _H, _K = 256, 256, 8
_R1 = 256
_R3 = 256
_NC, _NS = 2, 16
_NW = _NC * _NS
_GCHUNK = 128
_CW = 640


def _dot_t(x, w):
    return lax.dot_general(x, w, (((1,), (1,)), ((), ())),
                           preferred_element_type=jnp.float32)


def _stage1_body(cenall_ref, of_ref, siz_ref, lang_ref,
                 w1a_ref, w1b_ref, wgc_ref, wgs_ref, w1l_ref, b1_ref,
                 idx_ref, gidxt_ref, a_ref, c_ref):
    b = pl.program_id(0)
    i = pl.program_id(1)
    cen = cenall_ref[0, pl.ds(i * _R1, _R1), :]
    cenT = jnp.transpose(cenall_ref[0], (1, 0))
    d = jnp.zeros((_R1, _N), jnp.float32)
    for c in range(3):
        diff = cen[:, c:c + 1] - cenT[c:c + 1, :]
        d = d + diff * diff
    d = d / 25.0
    rowg = i * _R1 + lax.broadcasted_iota(jnp.int32, (_R1, _N), 0)
    colid_i = lax.broadcasted_iota(jnp.int32, (_R1, _N), 1)
    d = jnp.where(rowg == colid_i, jnp.inf, d)
    colid = colid_i.astype(jnp.float32)
    sels = []
    for k in range(_K):
        m = jnp.min(d, axis=1, keepdims=True)
        sel = jnp.min(jnp.where(d == m, colid, jnp.float32(_N)),
                      axis=1, keepdims=True)
        sels.append(sel)
        d = jnp.where(colid == sel, jnp.inf, d)
    idxf = jnp.concatenate(sels, axis=1)
    idx_ref[0] = idxf.astype(jnp.int32)
    gidxt_ref[...] = (jnp.transpose(idxf, (1, 0))
                      + jnp.float32(b * _N)).astype(jnp.int32)

    of = of_ref[0]
    g = (_dot_t(cen, wgc_ref[...]) * (1.0 / (5.0 + 1e-06))
         + _dot_t(siz_ref[0], wgs_ref[...]) * 0.5)
    langp = _dot_t(lang_ref[0], w1l_ref[...])
    a_ref[0] = _dot_t(of, w1a_ref[...]) + langp + b1_ref[...] + g
    cvals = _dot_t(of, w1b_ref[...]) - g
    u = lax.bitcast_convert_type(cvals, jnp.uint32)
    rb = (u + jnp.uint32(0x7FFF) + ((u >> 16) & jnp.uint32(1))) >> 16
    c_ref[0] = (rb[:, :_H // 2] | (rb[:, _H // 2:] << 16)).astype(jnp.uint32)


def _stage1(cen, of, siz, lang, w1a, w1b, wgc, wgs, w1l, b1):
    grid = (_B, _N // _R1)
    return pl.pallas_call(
        _stage1_body,
        grid=grid,
        in_specs=[
            pl.BlockSpec((1, _N, 3), lambda b, i: (b, 0, 0)),
            pl.BlockSpec((1, _R1, _D), lambda b, i: (b, i, 0)),
            pl.BlockSpec((1, _R1, 3), lambda b, i: (b, i, 0)),
            pl.BlockSpec((1, 1, _DL), lambda b, i: (b, 0, 0)),
            pl.BlockSpec((_H, _D), lambda b, i: (0, 0)),
            pl.BlockSpec((_H, _D), lambda b, i: (0, 0)),
            pl.BlockSpec((_H, 3), lambda b, i: (0, 0)),
            pl.BlockSpec((_H, 3), lambda b, i: (0, 0)),
            pl.BlockSpec((_H, _DL), lambda b, i: (0, 0)),
            pl.BlockSpec((1, _H), lambda b, i: (0, 0)),
        ],
        out_specs=[
            pl.BlockSpec((1, _R1, _K), lambda b, i: (b, i, 0)),
            pl.BlockSpec((_K, _R1), lambda b, i: (0, b * (_N // _R1) + i)),
            pl.BlockSpec((1, _R1, _H), lambda b, i: (b, i, 0)),
            pl.BlockSpec((1, _R1, _H // 2), lambda b, i: (b, i, 0)),
        ],
        out_shape=[
            jax.ShapeDtypeStruct((_B, _N, _K), jnp.int32),
            jax.ShapeDtypeStruct((_K, _B * _N), jnp.int32),
            jax.ShapeDtypeStruct((_B, _N, _H), jnp.float32),
            jax.ShapeDtypeStruct((_B, _N, _H // 2), jnp.uint32),
        ],
    )(cen, of, siz, lang, w1a, w1b, wgc, wgs, w1l, b1)


def _sc_gather(gidxt, c_all):
    n_idx = _B * _N * _K
    per_w = n_idx // _NW
    n_chunks = per_w // _GCHUNK

    mesh = plsc.VectorSubcoreMesh(core_axis_name="c", subcore_axis_name="s")

    @functools.partial(
        pl.kernel,
        mesh=mesh,
        out_type=jax.ShapeDtypeStruct((n_idx, _H // 2), jnp.uint32),
        scratch_types=[
            pltpu.VMEM((per_w,), jnp.int32),
            pltpu.VMEM((2, _GCHUNK, _H // 2), jnp.uint32),
            pltpu.SemaphoreType.DMA,
        ],
    )
    def k(gidx_hbm, c_hbm, out_hbm, idx_v, rows_v, gsem):
        wid = lax.axis_index("s") * _NC + lax.axis_index("c")
        base0 = wid * per_w
        pltpu.sync_copy(
            gidx_hbm.at[wid // _B, pl.ds((wid % _B) * per_w, per_w)], idx_v)
        cps = [None, None]
        cps[0] = pltpu.async_copy(
            c_hbm.at[idx_v.at[pl.ds(0, _GCHUNK)]], rows_v.at[0], gsem)
        for t in range(n_chunks):
            s = t % 2
            if t + 1 < n_chunks:
                cps[1 - s] = pltpu.async_copy(
                    c_hbm.at[idx_v.at[pl.ds((t + 1) * _GCHUNK, _GCHUNK)]],
                    rows_v.at[1 - s], gsem)
            cps[s].wait()
            pltpu.sync_copy(
                rows_v.at[s], out_hbm.at[pl.ds(base0 + t * _GCHUNK, _GCHUNK)])

    return k(gidxt, c_all)


def _stage3_body(a_ref, cg_ref, idx_ref, offull_ref, w2_ref, b2_ref,
                 out_ref, w_ref):
    i = pl.program_id(1)
    a = a_ref[0]
    cols = []
    for k in range(_K):
        pk = cg_ref[k]
        lo = lax.bitcast_convert_type(pk << 16, jnp.float32)
        hi = lax.bitcast_convert_type(pk & jnp.uint32(0xFFFF0000), jnp.float32)
        cgk = jnp.concatenate([lo, hi], axis=1)
        h = jnp.maximum(a + cgk, 0.0)
        cols.append(_dot_t(h, w2_ref[...]))
    scores = jnp.concatenate(cols, axis=1) + b2_ref[...]
    m = jnp.max(scores, axis=1, keepdims=True)
    e = jnp.exp(scores - m)
    w = e / jnp.sum(e, axis=1, keepdims=True)
    idx = idx_ref[0]
    colid = lax.broadcasted_iota(jnp.int32, (_R3, _N), 1)
    wmat = jnp.zeros((_R3, _N), jnp.float32)
    for k in range(_K):
        wmat = wmat + jnp.where(colid == idx[:, k:k + 1], w[:, k:k + 1], 0.0)
    ctx = jnp.dot(wmat, offull_ref[0], preferred_element_type=jnp.float32)
    out_ref[0] = offull_ref[0, pl.ds(i * _R3, _R3), :] + ctx
    w_ref[0] = w


def _stage3(a_all, cg3, idx, of, w2, b2):
    grid = (_B, _N // _R3)
    nb = _B * _N // _R3
    return pl.pallas_call(
        _stage3_body,
        grid=grid,
        in_specs=[
            pl.BlockSpec((1, _R3, _H), lambda b, i: (b, i, 0)),
            pl.BlockSpec((_K, _R3, _H // 2),
                         lambda b, i: (0, b * (_N // _R3) + i, 0)),
            pl.BlockSpec((1, _R3, _K), lambda b, i: (b, i, 0)),
            pl.BlockSpec((1, _N, _D), lambda b, i: (b, 0, 0)),
            pl.BlockSpec((1, _H), lambda b, i: (0, 0)),
            pl.BlockSpec((1, _K), lambda b, i: (0, 0)),
        ],
        out_specs=[
            pl.BlockSpec((1, _R3, _D), lambda b, i: (b, i, 0)),
            pl.BlockSpec((1, _R3, _K), lambda b, i: (b, i, 0)),
        ],
        out_shape=[
            jax.ShapeDtypeStruct((_B, _N, _D), jnp.float32),
            jax.ShapeDtypeStruct((_B, _N, _K), jnp.float32),
        ],
    )(a_all, cg3, idx, of, w2, b2)


def kernel(object_features, language_embedding, centers, sizes, W1, b1, W2, b2):
    w1a = W1[:, :_D]
    w1b = W1[:, _D:2 * _D]
    wgc = W1[:, 2 * _D:2 * _D + 3]
    wgs = W1[:, 2 * _D + 3:2 * _D + 6]
    w1l = W1[:, 2 * _D + 6:]
    b1r = b1.reshape(1, _H)
    b2r = jnp.broadcast_to(b2.reshape(1, 1), (1, _K))

    idx, gidxt, a_all, c_all = _stage1(
        centers, object_features, sizes,
        language_embedding.reshape(_B, 1, _DL),
        w1a, w1b, wgc, wgs, w1l, b1r)

    cg = _sc_gather(gidxt, c_all.reshape(_B * _N, _H // 2))

    out, w = _stage3(
        a_all, cg.reshape(_K, _B * _N, _H // 2), idx, object_features, W2, b2r)

    return (out, w, idx)


# speedup vs baseline: 18.5520x; 1.0122x over previous
import functools

import jax
import jax.numpy as jnp
from jax import lax
from jax.experimental import pallas as pl
from jax.experimental.pallas import tpu as pltpu
from jax.experimental.pallas import tpu_sc as plsc

_B, _N, _D = 4, 1024, 320
_DL, _H, _K = 256, 256, 8
_R1 = 256
_R3 = 256
_NC, _NS = 2, 16
_NW = _NC * _NS
_GCHUNK = 128
_CW = 640


def _dot_t(x, w):
    return lax.dot_general(x, w, (((1,), (1,)), ((), ())),
                           preferred_element_type=jnp.float32)


def _stage1_body(cenall_ref, of_ref, siz_ref, lang_ref,
                 w1a_ref, w1b_ref, wgc_ref, wgs_ref, w1l_ref, b1_ref,
                 idx_ref, gidxt_ref, a_ref, c_ref):
    b = pl.program_id(0)
    i = pl.program_id(1)
    cen = cenall_ref[0, pl.ds(i * _R1, _R1), :]
    cenT = jnp.transpose(cenall_ref[0], (1, 0))
    d = jnp.zeros((_R1, _N), jnp.float32)
    for c in range(3):
        diff = cen[:, c:c + 1] - cenT[c:c + 1, :]
        d = d + diff * diff
    d = d / 25.0
    rowg = i * _R1 + lax.broadcasted_iota(jnp.int32, (_R1, _N), 0)
    colid_i = lax.broadcasted_iota(jnp.int32, (_R1, _N), 1)
    d = jnp.where(rowg == colid_i, jnp.inf, d)
    colid = colid_i.astype(jnp.float32)
    sels = []
    for k in range(_K):
        m = jnp.min(d, axis=1, keepdims=True)
        sel = jnp.min(jnp.where(d == m, colid, jnp.float32(_N)),
                      axis=1, keepdims=True)
        sels.append(sel)
        d = jnp.where(colid == sel, jnp.inf, d)
    idxf = jnp.concatenate(sels, axis=1)
    idx_ref[0] = idxf.astype(jnp.int32)
    gidxt_ref[...] = (jnp.transpose(idxf, (1, 0))
                      + jnp.float32(b * _N)).astype(jnp.int32)

    of = of_ref[0]
    g = (_dot_t(cen, wgc_ref[...]) * (1.0 / (5.0 + 1e-06))
         + _dot_t(siz_ref[0], wgs_ref[...]) * 0.5)
    langp = _dot_t(lang_ref[0], w1l_ref[...])
    a_ref[0] = _dot_t(of, w1a_ref[...]) + langp + b1_ref[...] + g
    cvals = _dot_t(of, w1b_ref[...]) - g
    u = lax.bitcast_convert_type(cvals, jnp.uint32)
    rb = (u + jnp.uint32(0x7FFF) + ((u >> 16) & jnp.uint32(1))) >> 16
    c_ref[0] = (rb[:, :_H // 2] | (rb[:, _H // 2:] << 16)).astype(jnp.uint32)


def _stage1(cen, of, siz, lang, w1a, w1b, wgc, wgs, w1l, b1):
    grid = (_B, _N // _R1)
    return pl.pallas_call(
        _stage1_body,
        grid=grid,
        in_specs=[
            pl.BlockSpec((1, _N, 3), lambda b, i: (b, 0, 0)),
            pl.BlockSpec((1, _R1, _D), lambda b, i: (b, i, 0)),
            pl.BlockSpec((1, _R1, 3), lambda b, i: (b, i, 0)),
            pl.BlockSpec((1, 1, _DL), lambda b, i: (b, 0, 0)),
            pl.BlockSpec((_H, _D), lambda b, i: (0, 0)),
            pl.BlockSpec((_H, _D), lambda b, i: (0, 0)),
            pl.BlockSpec((_H, 3), lambda b, i: (0, 0)),
            pl.BlockSpec((_H, 3), lambda b, i: (0, 0)),
            pl.BlockSpec((_H, _DL), lambda b, i: (0, 0)),
            pl.BlockSpec((1, _H), lambda b, i: (0, 0)),
        ],
        out_specs=[
            pl.BlockSpec((1, _R1, _K), lambda b, i: (b, i, 0)),
            pl.BlockSpec((_K, _R1), lambda b, i: (0, b * (_N // _R1) + i)),
            pl.BlockSpec((1, _R1, _H), lambda b, i: (b, i, 0)),
            pl.BlockSpec((1, _R1, _H // 2), lambda b, i: (b, i, 0)),
        ],
        out_shape=[
            jax.ShapeDtypeStruct((_B, _N, _K), jnp.int32),
            jax.ShapeDtypeStruct((_K, _B * _N), jnp.int32),
            jax.ShapeDtypeStruct((_B, _N, _H), jnp.float32),
            jax.ShapeDtypeStruct((_B, _N, _H // 2), jnp.uint32),
        ],
    )(cen, of, siz, lang, w1a, w1b, wgc, wgs, w1l, b1)


def _sc_gather(gidxt, c_all):
    n_idx = _B * _N * _K
    per_w = n_idx // _NW
    n_chunks = per_w // _GCHUNK

    mesh = plsc.VectorSubcoreMesh(core_axis_name="c", subcore_axis_name="s")

    @functools.partial(
        pl.kernel,
        mesh=mesh,
        out_type=jax.ShapeDtypeStruct((n_idx, _H // 2), jnp.uint32),
        scratch_types=[
            pltpu.VMEM((per_w,), jnp.int32),
            pltpu.VMEM((2, _GCHUNK, _H // 2), jnp.uint32),
            pltpu.SemaphoreType.DMA,
        ],
    )
    def k(gidx_hbm, c_hbm, out_hbm, idx_v, rows_v, gsem):
        wid = lax.axis_index("s") * _NC + lax.axis_index("c")
        base0 = wid * per_w
        pltpu.sync_copy(
            gidx_hbm.at[wid // _B, pl.ds((wid % _B) * per_w, per_w)], idx_v)
        cps = [None, None]
        cps[0] = pltpu.async_copy(
            c_hbm.at[idx_v.at[pl.ds(0, _GCHUNK)]], rows_v.at[0], gsem)
        for t in range(n_chunks):
            s = t % 2
            if t + 1 < n_chunks:
                cps[1 - s] = pltpu.async_copy(
                    c_hbm.at[idx_v.at[pl.ds((t + 1) * _GCHUNK, _GCHUNK)]],
                    rows_v.at[1 - s], gsem)
            cps[s].wait()
            pltpu.sync_copy(
                rows_v.at[s], out_hbm.at[pl.ds(base0 + t * _GCHUNK, _GCHUNK)])

    return k(gidxt, c_all)


def _stage3_body(a_ref, cg_ref, idx_ref, offull_ref, w2_ref, b2_ref,
                 out_ref, w_ref):
    i = pl.program_id(1)
    a = a_ref[0]
    cols = []
    for k in range(_K):
        pk = cg_ref[k]
        lo = lax.bitcast_convert_type(pk << 16, jnp.float32)
        hi = lax.bitcast_convert_type(pk & jnp.uint32(0xFFFF0000), jnp.float32)
        cgk = jnp.concatenate([lo, hi], axis=1)
        h = jnp.maximum(a + cgk, 0.0)
        cols.append(_dot_t(h, w2_ref[...]))
    scores = jnp.concatenate(cols, axis=1) + b2_ref[...]
    m = jnp.max(scores, axis=1, keepdims=True)
    e = jnp.exp(scores - m)
    w = e / jnp.sum(e, axis=1, keepdims=True)
    idx = idx_ref[0]
    colid = lax.broadcasted_iota(jnp.int32, (_R3, _N), 1)
    wmat = jnp.zeros((_R3, _N), jnp.float32)
    for k in range(_K):
        wmat = jnp.where(colid == idx[:, k:k + 1], w[:, k:k + 1], wmat)
    ctx = jnp.dot(wmat, offull_ref[0], preferred_element_type=jnp.float32)
    out_ref[0] = offull_ref[0, pl.ds(i * _R3, _R3), :] + ctx
    w_ref[0] = w


def _stage3(a_all, cg3, idx, of, w2, b2):
    grid = (_B, _N // _R3)
    nb = _B * _N // _R3
    return pl.pallas_call(
        _stage3_body,
        grid=grid,
        in_specs=[
            pl.BlockSpec((1, _R3, _H), lambda b, i: (b, i, 0)),
            pl.BlockSpec((_K, _R3, _H // 2),
                         lambda b, i: (0, b * (_N // _R3) + i, 0)),
            pl.BlockSpec((1, _R3, _K), lambda b, i: (b, i, 0)),
            pl.BlockSpec((1, _N, _D), lambda b, i: (b, 0, 0)),
            pl.BlockSpec((1, _H), lambda b, i: (0, 0)),
            pl.BlockSpec((1, _K), lambda b, i: (0, 0)),
        ],
        out_specs=[
            pl.BlockSpec((1, _R3, _D), lambda b, i: (b, i, 0)),
            pl.BlockSpec((1, _R3, _K), lambda b, i: (b, i, 0)),
        ],
        out_shape=[
            jax.ShapeDtypeStruct((_B, _N, _D), jnp.float32),
            jax.ShapeDtypeStruct((_B, _N, _K), jnp.float32),
        ],
    )(a_all, cg3, idx, of, w2, b2)


def kernel(object_features, language_embedding, centers, sizes, W1, b1, W2, b2):
    w1a = W1[:, :_D]
    w1b = W1[:, _D:2 * _D]
    wgc = W1[:, 2 * _D:2 * _D + 3]
    wgs = W1[:, 2 * _D + 3:2 * _D + 6]
    w1l = W1[:, 2 * _D + 6:]
    b1r = b1.reshape(1, _H)
    b2r = jnp.broadcast_to(b2.reshape(1, 1), (1, _K))

    idx, gidxt, a_all, c_all = _stage1(
        centers, object_features, sizes,
        language_embedding.reshape(_B, 1, _DL),
        w1a, w1b, wgc, wgs, w1l, b1r)

    cg = _sc_gather(gidxt, c_all.reshape(_B * _N, _H // 2))

    out, w = _stage3(
        a_all, cg.reshape(_K, _B * _N, _H // 2), idx, object_features, W2, b2r)

    return (out, w, idx)
